# baseline (speedup 1.0000x reference)
import functools

import jax
import jax.numpy as jnp
from jax import lax
from jax.experimental import pallas as pl
from jax.experimental.pallas import tpu as pltpu
from jax.experimental.pallas import tpu_sc as plsc

_B = 16384
_F = 32
_L = 16
_NW = 32
_BPW = _B // _NW
_CHUNK = 128
_NCH = _BPW // _CHUNK
_V = 1000000

_mesh = plsc.VectorSubcoreMesh(core_axis_name="c", subcore_axis_name="s")


@functools.partial(
    pl.kernel,
    out_type=jax.ShapeDtypeStruct((_NW, _L), jnp.float32),
    mesh=_mesh,
    scratch_types=[
        pltpu.VMEM((_NCH, _CHUNK), jnp.int32),
        pltpu.VMEM((_NCH, _CHUNK), jnp.int32),
        pltpu.VMEM((_F * _BPW,), jnp.float32),
        pltpu.VMEM((_F, _BPW), jnp.float32),
        pltpu.VMEM((_L,), jnp.float32),
        pltpu.SemaphoreType.DMA,
        pltpu.SemaphoreType.DMA,
    ],
    compiler_params=pltpu.CompilerParams(
        disable_bounds_checks=True, use_tc_tiling_on_sc=False),
)
def _center_loss_partials(x_hbm, lab_hbm, cen_hbm, out_hbm,
                          lab_v, e_v, g_v, x_v, acc_v, gsem, xsem):
    wid = lax.axis_index("s") * 2 + lax.axis_index("c")

    pltpu.sync_copy(lab_hbm.at[pl.ds(wid * _NCH, _NCH)], lab_v)
    xcopy = pltpu.async_copy(
        x_hbm.at[:, pl.ds(wid * _BPW, _BPW)], x_v, xsem)

    copies = []
    for f in range(_F):
        row = cen_hbm.at[f]
        for j in range(_NCH):
            copies.append(
                pltpu.async_copy(
                    row.at[lab_v.at[j]],
                    g_v.at[pl.ds((f * _NCH + j) * _CHUNK, _CHUNK)],
                    gsem,
                )
            )
    xcopy.wait()
    for c in copies:
        c.wait()

    zero = jnp.zeros((_L,), jnp.float32)

    def body(f, acc):
        for o in range(0, _BPW, _L):
            d = x_v[f, pl.ds(o, _L)] - g_v[pl.ds(f * _BPW + o, _L)]
            acc = acc + d * d
        return acc

    acc_v[...] = lax.fori_loop(0, _F, body, zero)
    pltpu.sync_copy(acc_v, out_hbm.at[wid])


def kernel(x, labels, centers):
    labels2 = labels.astype(jnp.int32).reshape(_NW * _NCH, _CHUNK)
    partials = _center_loss_partials(x.T, labels2, centers.T)
    return jnp.sum(partials) / (2.0 * _B)


# speedup vs baseline: 1.0007x; 1.0007x over previous
import functools

import jax
import jax.numpy as jnp
from jax import lax
from jax.experimental import pallas as pl
from jax.experimental.pallas import tpu as pltpu
from jax.experimental.pallas import tpu_sc as plsc

_B = 16384
_F = 32
_L = 16
_NW = 32
_BPW = _B // _NW
_CHUNK = 128
_NCH = _BPW // _CHUNK
_V = 1000000

_mesh = plsc.VectorSubcoreMesh(core_axis_name="c", subcore_axis_name="s")


@functools.partial(
    pl.kernel,
    out_type=jax.ShapeDtypeStruct((_NW, _L), jnp.float32),
    mesh=_mesh,
    scratch_types=[
        pltpu.VMEM((_NCH, _CHUNK), jnp.int32),
        pltpu.VMEM((_NCH, _CHUNK), jnp.int32),
        pltpu.VMEM((_F * _BPW,), jnp.float32),
        pltpu.VMEM((_F, _BPW), jnp.float32),
        pltpu.VMEM((_L,), jnp.float32),
        pltpu.SemaphoreType.DMA,
        pltpu.SemaphoreType.DMA,
    ],
    compiler_params=pltpu.CompilerParams(
        disable_bounds_checks=True, use_tc_tiling_on_sc=False),
)
def _center_loss_partials(x_hbm, lab_hbm, cen_hbm, out_hbm,
                          lab_v, e_v, g_v, x_v, acc_v, gsem, xsem):
    wid = lax.axis_index("s") * 2 + lax.axis_index("c")

    pltpu.sync_copy(lab_hbm.at[pl.ds(wid * _NCH, _NCH)], lab_v)
    xcopy = pltpu.async_copy(
        x_hbm.at[:, pl.ds(wid * _BPW, _BPW)], x_v, xsem)

    copies = []
    for f in range(_F):
        row = cen_hbm.at[f]
        for j in range(_NCH):
            copies.append(
                pltpu.async_copy(
                    row.at[lab_v.at[j]],
                    g_v.at[pl.ds((f * _NCH + j) * _CHUNK, _CHUNK)],
                    gsem,
                )
            )
    xcopy.wait()
    for c in copies:
        c.wait()

    zero = jnp.zeros((_L,), jnp.float32)

    def body(f, acc):
        for o in range(0, _BPW, _L):
            d = x_v[f, pl.ds(o, _L)] - g_v[pl.ds(f * _BPW + o, _L)]
            acc = acc + d * d
        return acc

    acc_v[...] = lax.fori_loop(0, _F, body, zero)
    pltpu.sync_copy(acc_v, out_hbm.at[wid])


def kernel(x, labels, centers):
    labels2 = labels.astype(jnp.int32).reshape(_NW * _NCH, _CHUNK)
    cen_t = jax.lax.optimization_barrier(centers.T)
    x_t = jax.lax.optimization_barrier(x.T)
    partials = _center_loss_partials(x_t, labels2, cen_t)
    return jnp.sum(partials) / (2.0 * _B)


# speedup vs baseline: 4.9668x; 4.9632x over previous
import functools

import jax
import jax.numpy as jnp
from jax import lax
from jax.experimental import pallas as pl
from jax.experimental.pallas import tpu as pltpu
from jax.experimental.pallas import tpu_sc as plsc

_B = 16384
_D = 32
_L = 16
_NW = 32
_BPW = _B // _NW
_CHUNK = 128
_NCH = _BPW // _CHUNK

_mesh = plsc.VectorSubcoreMesh(core_axis_name="c", subcore_axis_name="s")


@functools.partial(
    pl.kernel,
    out_type=jax.ShapeDtypeStruct((_NW, _L), jnp.float32),
    mesh=_mesh,
    scratch_types=[
        pltpu.VMEM((_NCH, _CHUNK), jnp.int32),
        pltpu.VMEM((_BPW, _D), jnp.float32),
        pltpu.VMEM((_BPW, _D), jnp.float32),
        pltpu.VMEM((_L,), jnp.float32),
        pltpu.SemaphoreType.DMA,
        pltpu.SemaphoreType.DMA,
    ],
    compiler_params=pltpu.CompilerParams(use_tc_tiling_on_sc=False),
)
def _center_loss_partials(x_hbm, lab_hbm, cen_hbm, out_hbm,
                          idx_v, rows_v, x_v, acc_v, gsem, xsem):
    wid = lax.axis_index("s") * 2 + lax.axis_index("c")

    pltpu.sync_copy(lab_hbm.at[pl.ds(wid * _NCH, _NCH)], idx_v)

    copies = []
    for j in range(_NCH):
        copies.append(
            pltpu.async_copy(
                cen_hbm.at[idx_v.at[j]],
                rows_v.at[pl.ds(j * _CHUNK, _CHUNK)],
                gsem,
            )
        )
    xcopy = pltpu.async_copy(x_hbm.at[pl.ds(wid * _BPW, _BPW)], x_v, xsem)
    xcopy.wait()
    for c in copies:
        c.wait()

    zero = jnp.zeros((_L,), jnp.float32)

    def body(i, accs):
        a0, a1 = accs
        d0 = x_v[i, pl.ds(0, _L)] - rows_v[i, pl.ds(0, _L)]
        d1 = x_v[i, pl.ds(_L, _L)] - rows_v[i, pl.ds(_L, _L)]
        return (a0 + d0 * d0, a1 + d1 * d1)

    a0, a1 = lax.fori_loop(0, _BPW, body, (zero, zero))
    acc_v[...] = a0 + a1
    pltpu.sync_copy(acc_v, out_hbm.at[wid])


def kernel(x, labels, centers):
    labels2 = labels.astype(jnp.int32).reshape(_NW * _NCH, _CHUNK)
    partials = _center_loss_partials(x, labels2, centers)
    return jnp.sum(partials) / (2.0 * _B)
